# baseline (speedup 1.0000x reference)
import dataclasses
import functools
import math

import jax
import jax.numpy as jnp
from jax import lax
from jax.experimental import pallas as pl
from jax.experimental.pallas import tpu as pltpu
from jax.experimental.pallas import tpu_sc as plsc

_TOP_K = 32
_LANES = 16


def _kv_proj_body(mem_ref, wk_ref, bk_ref, wv_ref, bv_ref, k_ref, v_ref):
    m = mem_ref[0]
    k_ref[0] = jax.lax.dot_general(
        m, wk_ref[...], (((1,), (1,)), ((), ())),
        preferred_element_type=jnp.float32) + bk_ref[...]
    v_ref[0] = (jax.lax.dot_general(
        m, wv_ref[...], (((1,), (1,)), ((), ())),
        preferred_element_type=jnp.float32) + bv_ref[...]).astype(jnp.bfloat16)


def _scores_body(q_in_ref, wq_ref, bq_ref, k_ref, s_ref, *, scale):
    q = jax.lax.dot_general(
        q_in_ref[0], wq_ref[...], (((1,), (1,)), ((), ())),
        preferred_element_type=jnp.float32) + bq_ref[...]
    s_ref[0] = jax.lax.dot_general(
        q, k_ref[0], (((1,), (1,)), ((), ())),
        preferred_element_type=jnp.float32) * scale


def _attend_body(s_ref, thr_ref, v_ref, wo_ref, bo_ref, out_ref):
    s = s_ref[0]
    thr = thr_ref[0, 0]
    m = jnp.max(s, axis=-1, keepdims=True)
    p = jnp.where(s >= thr, jnp.exp(s - m), 0.0)
    z = jnp.sum(p, axis=-1, keepdims=True)
    ctx = jax.lax.dot_general(
        p.astype(jnp.bfloat16), v_ref[0], (((1,), (0,)), ((), ())),
        preferred_element_type=jnp.float32) / z
    out_ref[0] = jax.lax.dot_general(
        ctx.astype(jnp.bfloat16), wo_ref[...], (((1,), (1,)), ((), ())),
        preferred_element_type=jnp.float32) + bo_ref[...]


def _neg_inf16():
    return jnp.full((_LANES,), -jnp.inf, jnp.float32)


def _sc_topk_threshold(scores_flat, n_rows, row_len):
    info = plsc.get_sparse_core_info()
    nw = info.num_cores * info.num_subcores
    rows_per_w = n_rows // nw
    chunk_rows = 8
    n_chunks = rows_per_w // chunk_rows
    chunk_elems = chunk_rows * row_len
    nvec = row_len // _LANES
    mesh = plsc.VectorSubcoreMesh(core_axis_name="c", subcore_axis_name="s")

    def body(scores_hbm, thr_hbm, buf0, buf1, thr_v, sem0, sem1, *survs):
        wid = lax.axis_index("s") * info.num_cores + lax.axis_index("c")
        base_row = wid * rows_per_w
        base_elem = base_row * row_len

        def start(c, buf, sem):
            pltpu.make_async_copy(
                scores_hbm.at[pl.ds(base_elem + c * chunk_elems, chunk_elems)],
                buf, sem).start()

        def drain(buf, sem):
            pltpu.make_async_copy(
                scores_hbm.at[pl.ds(0, chunk_elems)], buf, sem).wait()

        def process_row(buf, local_row, out_slot):
            row_off = local_row * row_len
            unroll = 8
            n_outer = nvec // unroll

            def p1(j, carry):
                base = row_off + j * (unroll * _LANES)
                out = []
                for u in range(unroll):
                    m1, m2 = carry[2 * u], carry[2 * u + 1]
                    x = buf[pl.ds(base + u * _LANES, _LANES)]
                    t = jnp.minimum(m1, x)
                    out.append(jnp.maximum(m1, x))
                    out.append(jnp.maximum(m2, t))
                return tuple(out)

            accs = lax.fori_loop(
                0, n_outer, p1, tuple(_neg_inf16() for _ in range(2 * unroll)))
            m1, m2 = accs[0], accs[1]
            for u in range(1, unroll):
                a1, a2 = accs[2 * u], accs[2 * u + 1]
                m2 = jnp.maximum(jnp.minimum(m1, a1), jnp.maximum(m2, a2))
                m1 = jnp.maximum(m1, a1)
            lb = jnp.min(m2)

            def p2(j, off_vecs):
                base = row_off + j * (unroll * _LANES)
                out = []
                for u in range(unroll):
                    x = buf[pl.ds(base + u * _LANES, _LANES)]
                    msk = x >= lb
                    cs = jnp.cumsum(msk.astype(jnp.int32))
                    plsc.store_scatter(
                        survs[u], [off_vecs[u] + cs - 1], x, mask=msk)
                    out.append(
                        off_vecs[u] + plsc.all_reduce_population_count(msk))
                return tuple(out)

            off_vecs = lax.fori_loop(
                0, n_outer, p2,
                tuple(jnp.zeros((_LANES,), jnp.int32) for _ in range(unroll)))

            def merge(carry, xs_sorted):
                h1, h2 = carry
                u = lax.sort(jnp.maximum(h2, lax.rev(xs_sorted, (0,))))
                ur = lax.rev(u, (0,))
                return (lax.sort(jnp.maximum(h1, ur)),
                        lax.sort(jnp.minimum(h1, ur)))

            carry = (_neg_inf16(), _neg_inf16())
            for u in range(unroll):
                cnt_u = jnp.max(off_vecs[u])
                plsc.store_scatter(
                    survs[u], [cnt_u + lax.iota(jnp.int32, _LANES)],
                    _neg_inf16())

                def p3(i, c, _su=survs[u]):
                    return merge(c, lax.sort(_su[pl.ds(i * _LANES, _LANES)]))

                carry = lax.fori_loop(
                    0, (cnt_u + _LANES - 1) // _LANES, p3, carry)
            h2 = carry[1]
            plsc.store_scatter(
                thr_v, [jnp.full((_LANES,), out_slot, jnp.int32)], h2,
                mask=lax.iota(jnp.int32, _LANES) == 0)

        def process_chunk(c, buf):
            @pl.loop(0, chunk_rows)
            def _(r):
                process_row(buf, r, c * chunk_rows + r)

        start(0, buf0, sem0)
        start(1, buf1, sem1)

        @pl.loop(0, n_chunks // 2)
        def _(p):
            c0 = 2 * p
            drain(buf0, sem0)
            process_chunk(c0, buf0)

            @pl.when(c0 + 2 < n_chunks)
            def _():
                start(c0 + 2, buf0, sem0)

            drain(buf1, sem1)
            process_chunk(c0 + 1, buf1)

            @pl.when(c0 + 3 < n_chunks)
            def _():
                start(c0 + 3, buf1, sem1)

        pltpu.sync_copy(thr_v, thr_hbm.at[pl.ds(base_row, rows_per_w)])

    cp = pltpu.CompilerParams()
    if "needs_layout_passes" in pltpu.CompilerParams.__dataclass_fields__:
        cp = dataclasses.replace(cp, needs_layout_passes=False)
    call = pl.kernel(
        body,
        out_type=jax.ShapeDtypeStruct((n_rows,), jnp.float32),
        mesh=mesh,
        compiler_params=cp,
        scratch_types=[
            pltpu.VMEM((chunk_elems,), jnp.float32),
            pltpu.VMEM((chunk_elems,), jnp.float32),
            pltpu.VMEM((rows_per_w,), jnp.float32),
            pltpu.SemaphoreType.DMA,
            pltpu.SemaphoreType.DMA,
        ] + [pltpu.VMEM((row_len // 8 + _LANES,), jnp.float32)
             for _ in range(8)],
    )
    return call(scores_flat)


def kernel(query, mem_state, Wq, bq, Wk, bk, Wv, bv, Wo, bo):
    B, T, D = query.shape
    S = mem_state.shape[1]
    scale = 1.0 / math.sqrt(D)

    bq2, bk2, bv2, bo2 = (b.reshape(1, D) for b in (bq, bk, bv, bo))

    BS = min(1024, S)
    k_mat, v_mat = pl.pallas_call(
        _kv_proj_body,
        grid=(B, S // BS),
        in_specs=[
            pl.BlockSpec((1, BS, D), lambda b, s: (b, s, 0)),
            pl.BlockSpec((D, D), lambda b, s: (0, 0)),
            pl.BlockSpec((1, D), lambda b, s: (0, 0)),
            pl.BlockSpec((D, D), lambda b, s: (0, 0)),
            pl.BlockSpec((1, D), lambda b, s: (0, 0)),
        ],
        out_specs=[
            pl.BlockSpec((1, BS, D), lambda b, s: (b, s, 0)),
            pl.BlockSpec((1, BS, D), lambda b, s: (b, s, 0)),
        ],
        out_shape=[
            jax.ShapeDtypeStruct((B, S, D), jnp.float32),
            jax.ShapeDtypeStruct((B, S, D), jnp.bfloat16),
        ],
    )(mem_state, Wk, bk2, Wv, bv2)

    BT = min(256, T)
    scores = pl.pallas_call(
        functools.partial(_scores_body, scale=scale),
        grid=(B, T // BT),
        in_specs=[
            pl.BlockSpec((1, BT, D), lambda b, t: (b, t, 0)),
            pl.BlockSpec((D, D), lambda b, t: (0, 0)),
            pl.BlockSpec((1, D), lambda b, t: (0, 0)),
            pl.BlockSpec((1, S, D), lambda b, t: (b, 0, 0)),
        ],
        out_specs=pl.BlockSpec((1, BT, S), lambda b, t: (b, t, 0)),
        out_shape=jax.ShapeDtypeStruct((B, T, S), jnp.float32),
    )(query, Wq, bq2, k_mat)

    thr = _sc_topk_threshold(scores.reshape(-1), B * T, S)

    BA = min(256, T)
    thr4 = thr.reshape(B, T // BA, BA, 1)
    out = pl.pallas_call(
        _attend_body,
        grid=(B, T // BA),
        in_specs=[
            pl.BlockSpec((1, BA, S), lambda b, t: (b, t, 0)),
            pl.BlockSpec((1, 1, BA, 1), lambda b, t: (b, t, 0, 0)),
            pl.BlockSpec((1, S, D), lambda b, t: (b, 0, 0)),
            pl.BlockSpec((D, D), lambda b, t: (0, 0)),
            pl.BlockSpec((1, D), lambda b, t: (0, 0)),
        ],
        out_specs=pl.BlockSpec((1, BA, D), lambda b, t: (b, t, 0)),
        out_shape=jax.ShapeDtypeStruct((B, T, D), jnp.float32),
    )(scores, thr4, v_mat, Wo.astype(jnp.bfloat16), bo2)
    return out


# speedup vs baseline: 2.1759x; 2.1759x over previous
import dataclasses
import functools
import math

import jax
import jax.numpy as jnp
from jax import lax
from jax.experimental import pallas as pl
from jax.experimental.pallas import tpu as pltpu
from jax.experimental.pallas import tpu_sc as plsc

_TOP_K = 32
_LANES = 16


def _kv_proj_body(mem_ref, wk_ref, bk_ref, wv_ref, bv_ref, k_ref, v_ref):
    m = mem_ref[0]
    k_ref[0] = jax.lax.dot_general(
        m, wk_ref[...], (((1,), (1,)), ((), ())),
        preferred_element_type=jnp.float32) + bk_ref[...]
    v_ref[0] = (jax.lax.dot_general(
        m, wv_ref[...], (((1,), (1,)), ((), ())),
        preferred_element_type=jnp.float32) + bv_ref[...]).astype(jnp.bfloat16)


def _scores_body(q_in_ref, wq_ref, bq_ref, k_ref, s_ref, *, scale):
    q = jax.lax.dot_general(
        q_in_ref[0], wq_ref[...], (((1,), (1,)), ((), ())),
        preferred_element_type=jnp.float32) + bq_ref[...]
    s_ref[0] = jax.lax.dot_general(
        q, k_ref[0], (((1,), (1,)), ((), ())),
        preferred_element_type=jnp.float32) * scale


def _attend_body(s_ref, thr_ref, v_ref, wo_ref, bo_ref, out_ref):
    s = s_ref[0]
    thr = thr_ref[0, 0]
    m = jnp.max(s, axis=-1, keepdims=True)
    p = jnp.where(s >= thr, jnp.exp(s - m), 0.0)
    z = jnp.sum(p, axis=-1, keepdims=True)
    ctx = jax.lax.dot_general(
        p.astype(jnp.bfloat16), v_ref[0], (((1,), (0,)), ((), ())),
        preferred_element_type=jnp.float32) / z
    out_ref[0] = jax.lax.dot_general(
        ctx.astype(jnp.bfloat16), wo_ref[...], (((1,), (1,)), ((), ())),
        preferred_element_type=jnp.float32) + bo_ref[...]


def _neg_inf16():
    return jnp.full((_LANES,), -jnp.inf, jnp.float32)


def _sc_topk_threshold(scores_flat, n_rows, row_len):
    info = plsc.get_sparse_core_info()
    nw = info.num_cores * info.num_subcores
    rows_per_w = n_rows // nw
    chunk_rows = 8
    n_chunks = rows_per_w // chunk_rows
    chunk_elems = chunk_rows * row_len
    nvec = row_len // _LANES
    mesh = plsc.VectorSubcoreMesh(core_axis_name="c", subcore_axis_name="s")

    def body(scores_hbm, thr_hbm, buf0, buf1, surv, thr_v, sem0, sem1):
        wid = lax.axis_index("s") * info.num_cores + lax.axis_index("c")
        base_row = wid * rows_per_w
        base_elem = base_row * row_len

        def start(c, buf, sem):
            pltpu.make_async_copy(
                scores_hbm.at[pl.ds(base_elem + c * chunk_elems, chunk_elems)],
                buf, sem).start()

        def drain(buf, sem):
            pltpu.make_async_copy(
                scores_hbm.at[pl.ds(0, chunk_elems)], buf, sem).wait()

        def process_row(buf, local_row, out_slot):
            row_off = local_row * row_len
            unroll = 8
            n_outer = nvec // unroll

            def p1(j, carry):
                base = row_off + j * (unroll * _LANES)
                out = []
                for u in range(unroll):
                    m1, m2 = carry[2 * u], carry[2 * u + 1]
                    x = buf[pl.ds(base + u * _LANES, _LANES)]
                    t = jnp.minimum(m1, x)
                    out.append(jnp.maximum(m1, x))
                    out.append(jnp.maximum(m2, t))
                return tuple(out)

            accs = lax.fori_loop(
                0, n_outer, p1, tuple(_neg_inf16() for _ in range(2 * unroll)))
            m1, m2 = accs[0], accs[1]
            for u in range(1, unroll):
                a1, a2 = accs[2 * u], accs[2 * u + 1]
                m2 = jnp.maximum(jnp.minimum(m1, a1), jnp.maximum(m2, a2))
                m1 = jnp.maximum(m1, a1)
            lb = jnp.min(m2)

            lane_base = lax.iota(jnp.int32, _LANES) * nvec

            blk = 16

            def p2(j, pos):
                base = row_off + j * (blk * _LANES)
                xs, msks = [], []
                for u in range(blk):
                    x = buf[pl.ds(base + u * _LANES, _LANES)]
                    xs.append(x)
                    msks.append(x >= lb)
                poss = []
                for u in range(blk):
                    poss.append(pos)
                    pos = pos + msks[u].astype(jnp.int32)
                for u in range(blk):
                    plsc.store_scatter(surv, [poss[u]], xs[u], mask=msks[u])
                return pos

            pos = lax.fori_loop(0, nvec // blk, p2, lane_base)
            cnt_vec = pos - lane_base
            dmax = jnp.max(cnt_vec)

            def p3(d, carry):
                h1, h2 = carry
                x = plsc.load_gather(surv, [lane_base + d])
                xs = lax.sort(jnp.where(d < cnt_vec, x, -jnp.inf))
                u = lax.sort(jnp.maximum(h2, lax.rev(xs, (0,))))
                ur = lax.rev(u, (0,))
                return (lax.sort(jnp.maximum(h1, ur)),
                        lax.sort(jnp.minimum(h1, ur)))

            _, h2 = lax.fori_loop(0, dmax, p3, (_neg_inf16(), _neg_inf16()))
            plsc.store_scatter(
                thr_v, [jnp.full((_LANES,), out_slot, jnp.int32)], h2,
                mask=lax.iota(jnp.int32, _LANES) == 0)

        def process_chunk(c, buf):
            @pl.loop(0, chunk_rows)
            def _(r):
                process_row(buf, r, c * chunk_rows + r)

        start(0, buf0, sem0)
        start(1, buf1, sem1)

        @pl.loop(0, n_chunks // 2)
        def _(p):
            c0 = 2 * p
            drain(buf0, sem0)
            process_chunk(c0, buf0)

            @pl.when(c0 + 2 < n_chunks)
            def _():
                start(c0 + 2, buf0, sem0)

            drain(buf1, sem1)
            process_chunk(c0 + 1, buf1)

            @pl.when(c0 + 3 < n_chunks)
            def _():
                start(c0 + 3, buf1, sem1)

        pltpu.sync_copy(thr_v, thr_hbm.at[pl.ds(base_row, rows_per_w)])

    cp = pltpu.CompilerParams()
    if "needs_layout_passes" in pltpu.CompilerParams.__dataclass_fields__:
        cp = dataclasses.replace(cp, needs_layout_passes=False)
    call = pl.kernel(
        body,
        out_type=jax.ShapeDtypeStruct((n_rows,), jnp.float32),
        mesh=mesh,
        compiler_params=cp,
        scratch_types=[
            pltpu.VMEM((chunk_elems,), jnp.float32),
            pltpu.VMEM((chunk_elems,), jnp.float32),
            pltpu.VMEM((row_len,), jnp.float32),
            pltpu.VMEM((rows_per_w,), jnp.float32),
            pltpu.SemaphoreType.DMA,
            pltpu.SemaphoreType.DMA,
        ],
    )
    return call(scores_flat)


def kernel(query, mem_state, Wq, bq, Wk, bk, Wv, bv, Wo, bo):
    B, T, D = query.shape
    S = mem_state.shape[1]
    scale = 1.0 / math.sqrt(D)

    bq2, bk2, bv2, bo2 = (b.reshape(1, D) for b in (bq, bk, bv, bo))

    BS = min(1024, S)
    k_mat, v_mat = pl.pallas_call(
        _kv_proj_body,
        grid=(B, S // BS),
        in_specs=[
            pl.BlockSpec((1, BS, D), lambda b, s: (b, s, 0)),
            pl.BlockSpec((D, D), lambda b, s: (0, 0)),
            pl.BlockSpec((1, D), lambda b, s: (0, 0)),
            pl.BlockSpec((D, D), lambda b, s: (0, 0)),
            pl.BlockSpec((1, D), lambda b, s: (0, 0)),
        ],
        out_specs=[
            pl.BlockSpec((1, BS, D), lambda b, s: (b, s, 0)),
            pl.BlockSpec((1, BS, D), lambda b, s: (b, s, 0)),
        ],
        out_shape=[
            jax.ShapeDtypeStruct((B, S, D), jnp.float32),
            jax.ShapeDtypeStruct((B, S, D), jnp.bfloat16),
        ],
    )(mem_state, Wk, bk2, Wv, bv2)

    BT = min(256, T)
    scores = pl.pallas_call(
        functools.partial(_scores_body, scale=scale),
        grid=(B, T // BT),
        in_specs=[
            pl.BlockSpec((1, BT, D), lambda b, t: (b, t, 0)),
            pl.BlockSpec((D, D), lambda b, t: (0, 0)),
            pl.BlockSpec((1, D), lambda b, t: (0, 0)),
            pl.BlockSpec((1, S, D), lambda b, t: (b, 0, 0)),
        ],
        out_specs=pl.BlockSpec((1, BT, S), lambda b, t: (b, t, 0)),
        out_shape=jax.ShapeDtypeStruct((B, T, S), jnp.float32),
    )(query, Wq, bq2, k_mat)

    thr = _sc_topk_threshold(scores.reshape(-1), B * T, S)

    BA = min(256, T)
    thr4 = thr.reshape(B, T // BA, BA, 1)
    out = pl.pallas_call(
        _attend_body,
        grid=(B, T // BA),
        in_specs=[
            pl.BlockSpec((1, BA, S), lambda b, t: (b, t, 0)),
            pl.BlockSpec((1, 1, BA, 1), lambda b, t: (b, t, 0, 0)),
            pl.BlockSpec((1, S, D), lambda b, t: (b, 0, 0)),
            pl.BlockSpec((D, D), lambda b, t: (0, 0)),
            pl.BlockSpec((1, D), lambda b, t: (0, 0)),
        ],
        out_specs=pl.BlockSpec((1, BA, D), lambda b, t: (b, t, 0)),
        out_shape=jax.ShapeDtypeStruct((B, T, D), jnp.float32),
    )(scores, thr4, v_mat, Wo.astype(jnp.bfloat16), bo2)
    return out
